# Initial kernel scaffold; baseline (speedup 1.0000x reference)
import jax
import jax.numpy as jnp
from jax.experimental import pallas as pl


def kernel(x_nchw, w_oihw, bias):
    raise NotImplementedError("write your pallas kernel here")


# speedup vs baseline: 15.9842x; 15.9842x over previous
import functools

import jax
import jax.numpy as jnp
from jax import lax
from jax.experimental import pallas as pl
from jax.experimental.pallas import tpu as pltpu

_VMEM_LIMIT_BYTES = 48 * 1024 * 1024


def _conv_phase_kernel(ph_ref, w_ref, b_ref, o_ref, *, cin, wout, m):
    ee = ph_ref[0, 0 * cin:1 * cin, :]
    eo = ph_ref[0, 1 * cin:2 * cin, :]
    oe = ph_ref[0, 2 * cin:3 * cin, :]
    oo = ph_ref[0, 3 * cin:4 * cin, :]

    lane = lax.broadcasted_iota(jnp.int32, (1, m), 1)
    col0 = (lane % wout) == 0

    def shift_right(a, s):
        return jnp.concatenate(
            [jnp.zeros((cin, s), a.dtype), a[:, :m - s]], axis=-1)

    def mask_col0(a):
        return jnp.where(col0, jnp.zeros((), a.dtype), a)

    taps = (
        mask_col0(shift_right(oo, wout + 1)),
        shift_right(oe, wout),
        shift_right(oo, wout),
        mask_col0(shift_right(eo, 1)),
        ee,
        eo,
        mask_col0(shift_right(oo, 1)),
        oe,
        oo,
    )

    cout = b_ref.shape[0]
    acc = jnp.dot(w_ref[0:cout, :], taps[0],
                  preferred_element_type=jnp.float32)
    for t in range(1, 9):
        acc += jnp.dot(w_ref[t * cout:(t + 1) * cout, :], taps[t],
                       preferred_element_type=jnp.float32)
    o_ref[0] = acc + b_ref[...]


def kernel(x_nchw, w_oihw, bias):
    n, cin, h, w = x_nchw.shape
    cout = w_oihw.shape[0]
    hout, wout = h // 2, w // 2
    m = hout * wout

    x6 = x_nchw.reshape(n, cin, hout, 2, wout, 2)
    phases = jnp.stack(
        [x6[:, :, :, 0, :, 0],
         x6[:, :, :, 0, :, 1],
         x6[:, :, :, 1, :, 0],
         x6[:, :, :, 1, :, 1]],
        axis=1).reshape(n, 4 * cin, m).astype(jnp.bfloat16)

    w2 = jnp.transpose(w_oihw, (2, 3, 0, 1)).reshape(9 * cout, cin)
    w2 = w2.astype(jnp.bfloat16)
    b2 = bias.astype(jnp.float32).reshape(cout, 1)

    cost = pl.CostEstimate(
        flops=2 * n * m * 9 * cin * cout,
        transcendentals=0,
        bytes_accessed=phases.size * 2 + w2.size * 2 + n * cout * m * 4,
    )

    out = pl.pallas_call(
        functools.partial(_conv_phase_kernel, cin=cin, wout=wout, m=m),
        out_shape=jax.ShapeDtypeStruct((n, cout, m), jnp.float32),
        grid=(n,),
        in_specs=[
            pl.BlockSpec((1, 4 * cin, m), lambda i: (i, 0, 0)),
            pl.BlockSpec((9 * cout, cin), lambda i: (0, 0)),
            pl.BlockSpec((cout, 1), lambda i: (0, 0)),
        ],
        out_specs=pl.BlockSpec((1, cout, m), lambda i: (i, 0, 0)),
        compiler_params=pltpu.CompilerParams(
            dimension_semantics=("parallel",),
            vmem_limit_bytes=_VMEM_LIMIT_BYTES),
        cost_estimate=cost,
    )(phases, w2, b2)

    return out.reshape(n, cout, hout, wout).astype(x_nchw.dtype)


# speedup vs baseline: 18.5499x; 1.1605x over previous
import functools

import jax
import jax.numpy as jnp
from jax import lax
from jax.experimental import pallas as pl
from jax.experimental.pallas import tpu as pltpu

_VMEM_LIMIT_BYTES = 48 * 1024 * 1024


def _unpack_cols(packed_i32):
    even = lax.bitcast_convert_type(
        packed_i32.astype(jnp.int16), jnp.bfloat16)
    odd = lax.bitcast_convert_type(
        lax.shift_right_logical(packed_i32, jnp.int32(16)).astype(jnp.int16),
        jnp.bfloat16)
    return even, odd


def _conv_phase_kernel(xe_ref, xo_ref, w_ref, b_ref, o_ref, *, wout, m):
    cin = xe_ref.shape[1]
    ee, eo = _unpack_cols(xe_ref[0])
    oe, oo = _unpack_cols(xo_ref[0])

    lane = lax.broadcasted_iota(jnp.int32, (1, m), 1)
    col0 = (lane % wout) == 0

    def shift_right(a, s):
        return jnp.concatenate(
            [jnp.zeros((cin, s), a.dtype), a[:, :m - s]], axis=-1)

    def mask_col0(a):
        return jnp.where(col0, jnp.zeros((), a.dtype), a)

    taps = (
        mask_col0(shift_right(oo, wout + 1)),
        shift_right(oe, wout),
        shift_right(oo, wout),
        mask_col0(shift_right(eo, 1)),
        ee,
        eo,
        mask_col0(shift_right(oo, 1)),
        oe,
        oo,
    )

    cout = b_ref.shape[0]
    acc = jnp.dot(w_ref[0:cout, :], taps[0],
                  preferred_element_type=jnp.float32)
    for t in range(1, 9):
        acc += jnp.dot(w_ref[t * cout:(t + 1) * cout, :], taps[t],
                       preferred_element_type=jnp.float32)
    o_ref[0] = acc + b_ref[...]


def kernel(x_nchw, w_oihw, bias):
    n, cin, h, w = x_nchw.shape
    cout = w_oihw.shape[0]
    hout, wout = h // 2, w // 2
    m = hout * wout

    def pack(rows):
        pairs = rows.reshape(n, cin, hout, wout, 2)
        return lax.bitcast_convert_type(pairs, jnp.int32).reshape(n, cin, m)

    xe = pack(x_nchw[:, :, 0::2, :].astype(jnp.bfloat16))
    xo = pack(x_nchw[:, :, 1::2, :].astype(jnp.bfloat16))

    w2 = jnp.transpose(w_oihw, (2, 3, 0, 1)).reshape(9 * cout, cin)
    w2 = w2.astype(jnp.bfloat16)
    b2 = bias.astype(jnp.float32).reshape(cout, 1)

    cost = pl.CostEstimate(
        flops=2 * n * m * 9 * cin * cout,
        transcendentals=0,
        bytes_accessed=(xe.size + xo.size) * 4 + w2.size * 2 + n * cout * m * 4,
    )

    out = pl.pallas_call(
        functools.partial(_conv_phase_kernel, wout=wout, m=m),
        out_shape=jax.ShapeDtypeStruct((n, cout, m), jnp.float32),
        grid=(n,),
        in_specs=[
            pl.BlockSpec((1, cin, m), lambda i: (i, 0, 0)),
            pl.BlockSpec((1, cin, m), lambda i: (i, 0, 0)),
            pl.BlockSpec((9 * cout, cin), lambda i: (0, 0)),
            pl.BlockSpec((cout, 1), lambda i: (0, 0)),
        ],
        out_specs=pl.BlockSpec((1, cout, m), lambda i: (i, 0, 0)),
        compiler_params=pltpu.CompilerParams(
            dimension_semantics=("parallel",),
            vmem_limit_bytes=_VMEM_LIMIT_BYTES),
        cost_estimate=cost,
    )(xe, xo, w2, b2)

    return out.reshape(n, cout, hout, wout).astype(x_nchw.dtype)
